# baseline (device time: 36839 ns/iter reference)
import jax
import jax.numpy as jnp
from jax import lax
from jax.experimental import pallas as pl
from jax.experimental.pallas import tpu as pltpu

C = 8


def kernel(x):
    m_per, n = x.shape
    half = m_per // 2
    rc = half // C

    def body(x_ref, out_ref, x_send, x_recv, y_send, y_recv, local_sem):
        my_x = lax.axis_index("x")
        my_y = lax.axis_index("y")
        my_z = lax.axis_index("z")
        xp = (1 - my_x, my_y, my_z)
        yp = (my_x, 1 - my_y, my_z)

        barrier_sem = pltpu.get_barrier_semaphore()
        pl.semaphore_signal(
            barrier_sem, inc=1, device_id=xp,
            device_id_type=pl.DeviceIdType.MESH,
        )
        pl.semaphore_signal(
            barrier_sem, inc=1, device_id=yp,
            device_id_type=pl.DeviceIdType.MESH,
        )
        pl.semaphore_wait(barrier_sem, 2)

        my_half = my_y * half
        dst_base = my_x * m_per + my_half
        foreign = (1 - my_x) * m_per

        x_rdmas = []
        for c in range(C):
            r = pltpu.make_async_remote_copy(
                src_ref=x_ref.at[pl.ds(my_half + c * rc, rc), :],
                dst_ref=out_ref.at[pl.ds(dst_base + c * rc, rc), :],
                send_sem=x_send.at[c],
                recv_sem=x_recv.at[c],
                device_id=xp,
                device_id_type=pl.DeviceIdType.MESH,
            )
            r.start()
            x_rdmas.append(r)

        local_copy = pltpu.make_async_copy(
            x_ref, out_ref.at[pl.ds(my_x * m_per, m_per), :], local_sem
        )
        local_copy.start()

        y_rdmas = []
        for c in range(C):
            x_rdmas[c].wait_recv()
            fwd = foreign + my_half + c * rc
            r = pltpu.make_async_remote_copy(
                src_ref=out_ref.at[pl.ds(fwd, rc), :],
                dst_ref=out_ref.at[pl.ds(fwd, rc), :],
                send_sem=y_send.at[c],
                recv_sem=y_recv.at[c],
                device_id=yp,
                device_id_type=pl.DeviceIdType.MESH,
            )
            r.start()
            y_rdmas.append(r)

        for c in range(C):
            y_rdmas[c].wait_recv()
        for c in range(C):
            x_rdmas[c].wait_send()
            y_rdmas[c].wait_send()
        local_copy.wait()

    return pl.pallas_call(
        body,
        out_shape=jax.ShapeDtypeStruct((2 * m_per, n), x.dtype),
        in_specs=[pl.BlockSpec(memory_space=pltpu.VMEM)],
        out_specs=pl.BlockSpec(memory_space=pltpu.VMEM),
        scratch_shapes=[
            pltpu.SemaphoreType.DMA((C,)),
            pltpu.SemaphoreType.DMA((C,)),
            pltpu.SemaphoreType.DMA((C,)),
            pltpu.SemaphoreType.DMA((C,)),
            pltpu.SemaphoreType.DMA,
        ],
        compiler_params=pltpu.CompilerParams(collective_id=0),
    )(x)
